# Initial kernel scaffold; baseline (speedup 1.0000x reference)
import jax
import jax.numpy as jnp
from jax.experimental import pallas as pl


def kernel(indices, table):
    raise NotImplementedError("write your pallas kernel here")


# speedup vs baseline: 148.2418x; 148.2418x over previous
import functools

import jax
import jax.numpy as jnp
from jax import lax
from jax.experimental import pallas as pl
from jax.experimental.pallas import tpu as pltpu
from jax.experimental.pallas import tpu_sc as plsc

L = 16
NC = 2
NS = 16
NW = NC * NS


def _lookup_body(V, per_w, idx_hbm, table_hbm, out_hbm, table_v, in_v, out_v):
    wid = lax.axis_index("s") * NC + lax.axis_index("c")
    base = wid * per_w
    pltpu.sync_copy(table_hbm, table_v)
    pltpu.sync_copy(idx_hbm.at[pl.ds(base, per_w)], in_v)

    def step(i, carry):
        off = i * L
        idx = in_v[pl.ds(off, L)]
        valid = (idx >= 0) & (idx < V)
        safe = jnp.minimum(jnp.maximum(idx, 0), V - 1)
        vals = plsc.load_gather(table_v, [safe])
        out_v[pl.ds(off, L)] = jnp.where(valid, vals, jnp.int32(-1))
        return carry

    lax.fori_loop(0, per_w // L, step, 0)
    pltpu.sync_copy(out_v, out_hbm.at[pl.ds(base, per_w)])


def kernel(indices, table):
    B, F = indices.shape
    V = table.shape[0]
    N = B * F
    per_w = N // NW
    assert per_w * NW == N and per_w % L == 0
    tpad = ((V + L - 1) // L) * L
    table_p = jnp.pad(table, (0, tpad - V))
    flat = indices.reshape(N)

    mesh = plsc.VectorSubcoreMesh(core_axis_name="c", subcore_axis_name="s")
    body = functools.partial(_lookup_body, V, per_w)
    out = pl.kernel(
        body,
        mesh=mesh,
        compiler_params=pltpu.CompilerParams(needs_layout_passes=False),
        out_type=jax.ShapeDtypeStruct((N,), jnp.int32),
        scratch_types=[
            pltpu.VMEM((tpad,), jnp.int32),
            pltpu.VMEM((per_w,), jnp.int32),
            pltpu.VMEM((per_w,), jnp.int32),
        ],
    )(flat, table_p)
    return out.reshape(B, F)


# speedup vs baseline: 168.6481x; 1.1377x over previous
import functools

import jax
import jax.numpy as jnp
from jax import lax
from jax.experimental import pallas as pl
from jax.experimental.pallas import tpu as pltpu
from jax.experimental.pallas import tpu_sc as plsc

L = 16
NC = 2
NS = 16
NW = NC * NS


def _lookup_body(V, per_w, idx_hbm, table_hbm, out_hbm, table_v, in_v, out_v):
    wid = lax.axis_index("s") * NC + lax.axis_index("c")
    base = wid * per_w
    pltpu.sync_copy(table_hbm, table_v)
    pltpu.sync_copy(idx_hbm.at[pl.ds(base, per_w)], in_v)

    @plsc.parallel_loop(0, per_w, L, unroll=8)
    def _(off):
        idx = in_v[pl.ds(off, L)]
        valid = (idx >= 0) & (idx < V)
        safe = jnp.minimum(jnp.maximum(idx, 0), V - 1)
        vals = plsc.load_gather(table_v, [safe])
        out_v[pl.ds(off, L)] = jnp.where(valid, vals, jnp.int32(-1))
    pltpu.sync_copy(out_v, out_hbm.at[pl.ds(base, per_w)])


def kernel(indices, table):
    B, F = indices.shape
    V = table.shape[0]
    N = B * F
    per_w = N // NW
    assert per_w * NW == N and per_w % L == 0
    tpad = ((V + L - 1) // L) * L
    table_p = jnp.pad(table, (0, tpad - V))
    flat = indices.reshape(N)

    mesh = plsc.VectorSubcoreMesh(core_axis_name="c", subcore_axis_name="s")
    body = functools.partial(_lookup_body, V, per_w)
    out = pl.kernel(
        body,
        mesh=mesh,
        compiler_params=pltpu.CompilerParams(needs_layout_passes=False),
        out_type=jax.ShapeDtypeStruct((N,), jnp.int32),
        scratch_types=[
            pltpu.VMEM((tpad,), jnp.int32),
            pltpu.VMEM((per_w,), jnp.int32),
            pltpu.VMEM((per_w,), jnp.int32),
        ],
    )(flat, table_p)
    return out.reshape(B, F)


# speedup vs baseline: 248.4455x; 1.4732x over previous
import functools

import jax
import jax.numpy as jnp
from jax import lax
from jax.experimental import pallas as pl
from jax.experimental.pallas import tpu as pltpu
from jax.experimental.pallas import tpu_sc as plsc

L = 16
NC = 2
NS = 16
NW = NC * NS
CH = 128


def _lookup_body(V, cols, rows_per_w, idx_hbm, table_hbm, out_hbm,
                 table_v, in_v, out_v):
    wid = lax.axis_index("s") * NC + lax.axis_index("c")
    r0 = wid * rows_per_w
    pltpu.sync_copy(table_hbm, table_v)
    lane = lax.iota(jnp.int32, L)
    group = 4 * cols

    def chunk(c, carry):
        base = r0 + c * CH
        pltpu.sync_copy(idx_hbm.at[pl.ds(base, CH)], in_v)

        @plsc.parallel_loop(0, CH, 4)
        def _(grow):
            for j in range(group // L):
                p0 = j * L
                dr0, c0 = divmod(p0, cols)
                wb = min(L, cols - c0)
                crossed = lane >= wb
                row = grow + jnp.where(crossed, dr0 + 1, dr0)
                col = jnp.where(crossed, c0 - cols, c0) + lane
                idx = plsc.load_gather(in_v, [row, col])
                valid = (idx >= 0) & (idx < V)
                safe = jnp.minimum(jnp.maximum(idx, 0), V - 1)
                vals = plsc.load_gather(table_v, [safe])
                res = jnp.where(valid, vals, jnp.int32(-1))
                plsc.store_scatter(out_v, [row, col], res)

        pltpu.sync_copy(out_v, out_hbm.at[pl.ds(base, CH)])
        return carry

    lax.fori_loop(0, rows_per_w // CH, chunk, 0)


def kernel(indices, table):
    B, F = indices.shape
    V = table.shape[0]
    rows_per_w = B // NW
    assert rows_per_w * NW == B and rows_per_w % CH == 0
    tpad = ((V + L - 1) // L) * L
    table_p = jnp.pad(table, (0, tpad - V))

    mesh = plsc.VectorSubcoreMesh(core_axis_name="c", subcore_axis_name="s")
    body = functools.partial(_lookup_body, V, F, rows_per_w)
    return pl.kernel(
        body,
        mesh=mesh,
        compiler_params=pltpu.CompilerParams(needs_layout_passes=False),
        out_type=jax.ShapeDtypeStruct((B, F), jnp.int32),
        scratch_types=[
            pltpu.VMEM((tpad,), jnp.int32),
            pltpu.VMEM((CH, F), jnp.int32),
            pltpu.VMEM((CH, F), jnp.int32),
        ],
    )(indices, table_p)


# speedup vs baseline: 248.4635x; 1.0001x over previous
import functools

import jax
import jax.numpy as jnp
from jax import lax
from jax.experimental import pallas as pl
from jax.experimental.pallas import tpu as pltpu
from jax.experimental.pallas import tpu_sc as plsc

L = 16
NC = 2
NS = 16
NW = NC * NS
CH = 128


def _lookup_body(V, cols, rows_per_w, idx_hbm, table_hbm, out_hbm,
                 table_v, in_v, out_v):
    wid = lax.axis_index("s") * NC + lax.axis_index("c")
    r0 = wid * rows_per_w
    pltpu.sync_copy(table_hbm, table_v)
    lane = lax.iota(jnp.int32, L)
    group = 4 * cols

    def chunk(c, carry):
        base = r0 + c * CH
        pltpu.sync_copy(idx_hbm.at[pl.ds(base, CH)], in_v)

        @plsc.parallel_loop(0, CH, 4)
        def _(grow):
            for j in range(group // L):
                p0 = j * L
                dr0, c0 = divmod(p0, cols)
                wb = min(L, cols - c0)
                crossed = lane >= wb
                row = grow + jnp.where(crossed, dr0 + 1, dr0)
                col = jnp.where(crossed, c0 - cols, c0) + lane
                idx = plsc.load_gather(in_v, [row, col])
                valid = (idx >= 0) & (idx < V)
                safe = jnp.minimum(jnp.maximum(idx, 0), V - 1)
                vals = plsc.load_gather(table_v, [safe])
                res = jnp.where(valid, vals, jnp.int32(-1))
                plsc.store_scatter(out_v, [row, col], res)

        pltpu.sync_copy(out_v, out_hbm.at[pl.ds(base, CH)])
        return carry

    lax.fori_loop(0, rows_per_w // CH, chunk, 0)


def kernel(indices, table):
    B, F = indices.shape
    V = table.shape[0]
    rows_per_w = B // NW
    assert rows_per_w * NW == B and rows_per_w % CH == 0
    tpad = ((V + L - 1) // L) * L
    table_p = jnp.pad(table, (0, tpad - V))

    mesh = plsc.VectorSubcoreMesh(core_axis_name="c", subcore_axis_name="s")
    body = functools.partial(_lookup_body, V, F, rows_per_w)
    return pl.kernel(
        body,
        mesh=mesh,
        compiler_params=pltpu.CompilerParams(
            needs_layout_passes=False, use_tc_tiling_on_sc=True),
        out_type=jax.ShapeDtypeStruct((B, F), jnp.int32),
        scratch_types=[
            pltpu.VMEM((tpad,), jnp.int32),
            pltpu.VMEM((CH, F), jnp.int32),
            pltpu.VMEM((CH, F), jnp.int32),
        ],
    )(indices, table_p)


# speedup vs baseline: 439.7095x; 1.7697x over previous
import functools

import jax
import jax.numpy as jnp
from jax import lax
from jax.experimental import pallas as pl
from jax.experimental.pallas import tpu as pltpu
from jax.experimental.pallas import tpu_sc as plsc

L = 16
NC = 2
NS = 16
NW = NC * NS


def _lookup_body(V, rows, cols_per_w, idx_hbm, table_hbm, out_hbm,
                 table_v, in_v, out_v):
    wid = lax.axis_index("s") * NC + lax.axis_index("c")
    c0 = wid * cols_per_w
    pltpu.sync_copy(table_hbm, table_v)
    pltpu.sync_copy(idx_hbm.at[:, pl.ds(c0, cols_per_w)], in_v)
    wpr = cols_per_w // L

    @plsc.parallel_loop(0, rows * wpr, 1, unroll=8)
    def _(w):
        r = w // wpr
        c = (w % wpr) * L
        idx = in_v[r, pl.ds(c, L)]
        valid = (idx >= 0) & (idx < V)
        safe = jnp.minimum(jnp.maximum(idx, 0), V - 1)
        vals = plsc.load_gather(table_v, [safe])
        out_v[r, pl.ds(c, L)] = jnp.where(valid, vals, jnp.int32(-1))

    pltpu.sync_copy(out_v, out_hbm.at[:, pl.ds(c0, cols_per_w)])


def kernel(indices, table):
    B, F = indices.shape
    V = table.shape[0]
    cols_per_w = B // NW
    assert cols_per_w * NW == B and cols_per_w % L == 0
    tpad = ((V + L - 1) // L) * L
    table_p = jnp.pad(table, (0, tpad - V))
    idx_t = indices.T

    mesh = plsc.VectorSubcoreMesh(core_axis_name="c", subcore_axis_name="s")
    body = functools.partial(_lookup_body, V, F, cols_per_w)
    out_t = pl.kernel(
        body,
        mesh=mesh,
        compiler_params=pltpu.CompilerParams(
            needs_layout_passes=False, use_tc_tiling_on_sc=True),
        out_type=jax.ShapeDtypeStruct((F, B), jnp.int32),
        scratch_types=[
            pltpu.VMEM((tpad,), jnp.int32),
            pltpu.VMEM((F, cols_per_w), jnp.int32),
            pltpu.VMEM((F, cols_per_w), jnp.int32),
        ],
    )(idx_t, table_p)
    return out_t.T
